# baseline (speedup 1.0000x reference)
import functools

import jax
import jax.numpy as jnp
from jax import lax
from jax.experimental import pallas as pl
from jax.experimental.pallas import tpu as pltpu
from jax.experimental.pallas import tpu_sc as plsc

_NC = 2
_NS = 16
_NW = _NC * _NS
_L = 16

_N_ROWS = 16384
_N_COLS = 512
_K = 128
_KV = _K // _L

_SC_ROWS = 4096
_TC_ROWS = _N_ROWS - _SC_ROWS

_ROWS_PER_W = _SC_ROWS // _NW
_CHUNK = 64
_N_CHUNKS = _ROWS_PER_W // _CHUNK
_NBUF = 2
_N_STEPS = _N_CHUNKS // _NBUF
_UNROLL = 2

_TC_BLOCK = 2048


def _sc_body(x_hbm, idx_hbm, out_hbm, idx_v,
             xb0, xb1, ob0, ob1, is0, is1, os0, os1):
    wid = lax.axis_index("s") * _NC + lax.axis_index("c")
    base = wid * _ROWS_PER_W
    pltpu.sync_copy(idx_hbm, idx_v)
    idx_vecs = [idx_v[pl.ds(k * _L, _L)] for k in range(_KV)]
    xbufs, obufs = (xb0, xb1), (ob0, ob1)
    isems, osems = (is0, is1), (os0, os1)

    def in_copy(c, b):
        row0 = base + c * _CHUNK
        return pltpu.make_async_copy(
            x_hbm.at[pl.ds(row0, _CHUNK)], xbufs[b], isems[b])

    def out_copy(c, b):
        row0 = base + c * _CHUNK
        return pltpu.make_async_copy(
            obufs[b], out_hbm.at[pl.ds(row0, _CHUNK)], osems[b])

    for b in range(_NBUF):
        in_copy(b, b).start()

    def step(s, carry):
        for b in range(_NBUF):
            c = s * _NBUF + b
            in_copy(c, b).wait()

            @pl.when(s > 0)
            def _():
                out_copy(c - _NBUF, b).wait()

            ob = obufs[b]
            xbuf = xbufs[b]

            @plsc.parallel_loop(0, _CHUNK, step=1, unroll=_UNROLL)
            def _(r):
                rv = jnp.full((_L,), r, jnp.int32)
                for k in range(_KV):
                    v = plsc.load_gather(xbuf, [rv, idx_vecs[k]])
                    ob[r, pl.ds(k * _L, _L)] = v

            @pl.when(s < _N_STEPS - 1)
            def _():
                in_copy(c + _NBUF, b).start()

            out_copy(c, b).start()
        return carry

    lax.fori_loop(0, _N_STEPS, step, 0)

    for b in range(_NBUF):
        out_copy(_N_CHUNKS - _NBUF + b, b).wait()


@functools.partial(
    pl.kernel,
    out_type=jax.ShapeDtypeStruct((_SC_ROWS, _K), jnp.float32),
    mesh=plsc.VectorSubcoreMesh(core_axis_name="c", subcore_axis_name="s"),
    scratch_types=[
        pltpu.VMEM((_K,), jnp.int32),
        pltpu.VMEM((_CHUNK, _N_COLS), jnp.float32),
        pltpu.VMEM((_CHUNK, _N_COLS), jnp.float32),
        pltpu.VMEM((_CHUNK, _K), jnp.float32),
        pltpu.VMEM((_CHUNK, _K), jnp.float32),
        pltpu.SemaphoreType.DMA,
        pltpu.SemaphoreType.DMA,
        pltpu.SemaphoreType.DMA,
        pltpu.SemaphoreType.DMA,
    ],
    compiler_params=pltpu.CompilerParams(use_tc_tiling_on_sc=True,
                                         needs_layout_passes=False,
                                         skip_device_barrier=True),
)
def _sc_gather(x_hbm, idx_hbm, out_hbm, idx_v,
               xb0, xb1, ob0, ob1, is0, is1, os0, os1):
    _sc_body(x_hbm, idx_hbm, out_hbm, idx_v,
             xb0, xb1, ob0, ob1, is0, is1, os0, os1)


def _tc_gather_block(x_ref, idx_ref, out_ref):
    idx = idx_ref[0, :]
    col = jax.lax.broadcasted_iota(jnp.int32, (_N_COLS, _K), 0)
    onehot = (col == idx[None, :]).astype(jnp.float32)
    out_ref[...] = jnp.dot(x_ref[...], onehot,
                           preferred_element_type=jnp.float32)


def _tc_gather(x, indices2d):
    grid = (_TC_ROWS // _TC_BLOCK,)
    row0 = _SC_ROWS // _TC_BLOCK
    return pl.pallas_call(
        _tc_gather_block,
        grid=grid,
        in_specs=[
            pl.BlockSpec((_TC_BLOCK, _N_COLS), lambda i: (i + row0, 0)),
            pl.BlockSpec((1, _K), lambda i: (0, 0)),
        ],
        out_specs=pl.BlockSpec((_TC_BLOCK, _K), lambda i: (i, 0)),
        out_shape=jax.ShapeDtypeStruct((_TC_ROWS, _K), jnp.float32),
    )(x, indices2d)


def kernel(x, indices):
    out_sc = _sc_gather(x, indices)
    out_tc = _tc_gather(x, indices.reshape(1, _K))
    return jnp.concatenate([out_sc, out_tc], axis=0)


# speedup vs baseline: 1.0896x; 1.0896x over previous
import functools

import jax
import jax.numpy as jnp
from jax import lax
from jax.experimental import pallas as pl
from jax.experimental.pallas import tpu as pltpu
from jax.experimental.pallas import tpu_sc as plsc

_NC = 2
_NS = 16
_NW = _NC * _NS
_L = 16

_N_ROWS = 16384
_N_COLS = 512
_K = 128
_KV = _K // _L

_SC_ROWS = 4096
_TC_ROWS = _N_ROWS - _SC_ROWS

_ROWS_PER_W = _SC_ROWS // _NW
_CHUNK = 64
_N_CHUNKS = _ROWS_PER_W // _CHUNK
_NBUF = 2
_N_STEPS = _N_CHUNKS // _NBUF
_UNROLL = 2

_TC_BLOCK = 2048


def _sc_body(x_hbm, idx_hbm, out_hbm, idx_v,
             xb0, xb1, ob0, ob1, is0, is1, os0, os1):
    wid = lax.axis_index("s") * _NC + lax.axis_index("c")
    base = wid * _ROWS_PER_W
    pltpu.sync_copy(idx_hbm, idx_v)
    idx_vecs = [idx_v[pl.ds(k * _L, _L)] for k in range(_KV)]
    xbufs, obufs = (xb0, xb1), (ob0, ob1)
    isems, osems = (is0, is1), (os0, os1)

    def in_copy(c, b):
        row0 = base + c * _CHUNK
        return pltpu.make_async_copy(
            x_hbm.at[pl.ds(row0, _CHUNK)], xbufs[b], isems[b])

    def out_copy(c, b):
        row0 = base + c * _CHUNK
        return pltpu.make_async_copy(
            obufs[b], out_hbm.at[pl.ds(row0, _CHUNK)], osems[b])

    for b in range(_NBUF):
        in_copy(b, b).start()

    def step(s, carry):
        for b in range(_NBUF):
            c = s * _NBUF + b
            in_copy(c, b).wait()

            @pl.when(s > 0)
            def _():
                out_copy(c - _NBUF, b).wait()

            ob = obufs[b]
            xbuf = xbufs[b]

            @plsc.parallel_loop(0, _CHUNK, step=1, unroll=_UNROLL)
            def _(r):
                rv = jnp.full((_L,), r, jnp.int32)
                for k in range(_KV):
                    v = plsc.load_gather(xbuf, [rv, idx_vecs[k]])
                    ob[r, pl.ds(k * _L, _L)] = v

            @pl.when(s < _N_STEPS - 1)
            def _():
                in_copy(c + _NBUF, b).start()

            out_copy(c, b).start()
        return carry

    lax.fori_loop(0, _N_STEPS, step, 0)

    for b in range(_NBUF):
        out_copy(_N_CHUNKS - _NBUF + b, b).wait()


@functools.partial(
    pl.kernel,
    out_type=jax.ShapeDtypeStruct((_SC_ROWS, _K), jnp.float32),
    mesh=plsc.VectorSubcoreMesh(core_axis_name="c", subcore_axis_name="s"),
    scratch_types=[
        pltpu.VMEM((_K,), jnp.int32),
        pltpu.VMEM((_CHUNK, _N_COLS), jnp.float32),
        pltpu.VMEM((_CHUNK, _N_COLS), jnp.float32),
        pltpu.VMEM((_CHUNK, _K), jnp.float32),
        pltpu.VMEM((_CHUNK, _K), jnp.float32),
        pltpu.SemaphoreType.DMA,
        pltpu.SemaphoreType.DMA,
        pltpu.SemaphoreType.DMA,
        pltpu.SemaphoreType.DMA,
    ],
    compiler_params=pltpu.CompilerParams(use_tc_tiling_on_sc=True,
                                         needs_layout_passes=False),
)
def _sc_gather(x_hbm, idx_hbm, out_hbm, idx_v,
               xb0, xb1, ob0, ob1, is0, is1, os0, os1):
    _sc_body(x_hbm, idx_hbm, out_hbm, idx_v,
             xb0, xb1, ob0, ob1, is0, is1, os0, os1)


def _tc_gather_block(x_ref, idx_ref, out_ref):
    idx = idx_ref[0, :]
    col = jax.lax.broadcasted_iota(jnp.int32, (_N_COLS, _K), 0)
    onehot = (col == idx[None, :]).astype(jnp.float32)
    out_ref[...] = jnp.dot(x_ref[...], onehot,
                           preferred_element_type=jnp.float32)


def _tc_gather(x, indices2d):
    grid = (_TC_ROWS // _TC_BLOCK,)
    row0 = _SC_ROWS // _TC_BLOCK
    return pl.pallas_call(
        _tc_gather_block,
        grid=grid,
        in_specs=[
            pl.BlockSpec((_TC_BLOCK, _N_COLS), lambda i: (i + row0, 0)),
            pl.BlockSpec((1, _K), lambda i: (0, 0)),
        ],
        out_specs=pl.BlockSpec((_TC_BLOCK, _K), lambda i: (i + row0, 0)),
        out_shape=jax.ShapeDtypeStruct((_N_ROWS, _K), jnp.float32),
    )(x, indices2d)


def kernel(x, indices):
    out_sc = _sc_gather(x, indices)
    out_tc = _tc_gather(x, indices.reshape(1, _K))
    return lax.dynamic_update_slice(out_tc, out_sc, (0, 0))
